# baseline (speedup 1.0000x reference)
import functools

import jax
import jax.numpy as jnp
from jax.experimental import pallas as pl

_TOKENS = 8192
_BLK = 512


def _router_body(x_ref, u_ref, w1_ref, b1_ref, w2_ref, b2_ref, tab_ref,
                 shp_ref, idx_ref):
    x = x_ref[...]
    h = jnp.maximum(
        jnp.dot(x, w1_ref[...], preferred_element_type=jnp.float32)
        + b1_ref[...], 0.0)
    s = (jnp.dot(h, w2_ref[...], preferred_element_type=jnp.float32)
         + b2_ref[...])
    logits = jax.nn.sigmoid(s)
    u = jnp.clip(u_ref[...], 1e-10, 1.0 - 1e-10)
    g = -jnp.log(-jnp.log(u))
    score = logits + g
    m = jnp.max(score, axis=-1, keepdims=True)
    iota = jax.lax.broadcasted_iota(jnp.int32, score.shape, 1)
    idx = jnp.min(jnp.where(score == m, iota, score.shape[-1]),
                  axis=-1, keepdims=True)
    one_hot = (iota == idx).astype(jnp.float32)
    shp_ref[...] = jnp.dot(one_hot, tab_ref[...],
                           preferred_element_type=jnp.float32)
    idx_ref[...] = idx


@functools.partial(jax.jit, static_argnames=())
def kernel(state, gumbel_u, W1, b1, W2, b2, shape_table):
    tokens, obs_dim = state.shape
    hidden = W1.shape[1]
    n_arcs = W2.shape[1]
    tab_w = shape_table.shape[1]
    grid = (tokens // _BLK,)
    shp, idx2 = pl.pallas_call(
        _router_body,
        grid=grid,
        in_specs=[
            pl.BlockSpec((_BLK, obs_dim), lambda i: (i, 0)),
            pl.BlockSpec((_BLK, n_arcs), lambda i: (i, 0)),
            pl.BlockSpec((obs_dim, hidden), lambda i: (0, 0)),
            pl.BlockSpec((1, hidden), lambda i: (0, 0)),
            pl.BlockSpec((hidden, n_arcs), lambda i: (0, 0)),
            pl.BlockSpec((1, n_arcs), lambda i: (0, 0)),
            pl.BlockSpec((n_arcs, tab_w), lambda i: (0, 0)),
        ],
        out_specs=[
            pl.BlockSpec((_BLK, tab_w), lambda i: (i, 0)),
            pl.BlockSpec((_BLK, 1), lambda i: (i, 0)),
        ],
        out_shape=[
            jax.ShapeDtypeStruct((tokens, tab_w), jnp.float32),
            jax.ShapeDtypeStruct((tokens, 1), jnp.int32),
        ],
    )(state, gumbel_u, W1, b1.reshape(1, -1), W2, b2.reshape(1, -1),
      shape_table)
    return shp, idx2.reshape(tokens)
